# baseline (speedup 1.0000x reference)
import jax
import jax.numpy as jnp
from jax import lax
from jax.experimental import pallas as pl
from jax.experimental.pallas import tpu as pltpu

_N_CLASSES = 80
_HM = 128
_K = 100
_DOWN = 4.0
_SIZE = 512.0
_CHUNK = 8
_G = 2
_ROWS = _N_CLASSES * _HM
_NEG = float("-inf")
_BIG = 2 ** 30


def _nms_topk_body(y_ref, out_ref, *scratch):
    s_refs = scratch[:_G]
    sel_refs = scratch[_G:]
    rmv0 = []
    rmi0 = []
    for b in range(_G):
        rmv_parts = []
        rmi_parts = []
        for ci in range(_N_CLASSES // _CHUNK):
            x = y_ref[b, ci * _CHUNK:(ci + 1) * _CHUNK, :, :]
            pad_h = jnp.full((_CHUNK, 1, _HM), _NEG, jnp.float32)
            up = jnp.concatenate([x[:, 1:, :], pad_h], axis=1)
            dn = jnp.concatenate([pad_h, x[:, :-1, :]], axis=1)
            v = jnp.maximum(jnp.maximum(x, up), dn)
            pad_w = jnp.full((_CHUNK, _HM, 1), _NEG, jnp.float32)
            lf = jnp.concatenate([v[:, :, 1:], pad_w], axis=2)
            rt = jnp.concatenate([pad_w, v[:, :, :-1]], axis=2)
            hmax = jnp.maximum(jnp.maximum(v, lf), rt)
            sc = jnp.where(hmax == x, x, 0.0)
            s_refs[b][pl.ds(ci * _CHUNK * _HM, _CHUNK * _HM), :] = \
                sc.reshape(_CHUNK * _HM, _HM)
            rmv = jnp.max(sc, axis=2)
            w80 = lax.broadcasted_iota(jnp.int32, sc.shape, 2) * _N_CLASSES
            rmi_rel = jnp.min(
                jnp.where(sc == rmv[:, :, None], w80, _BIG), axis=2)
            offs = (lax.broadcasted_iota(jnp.int32, rmv.shape, 1)
                    * (_HM * _N_CLASSES)
                    + lax.broadcasted_iota(jnp.int32, rmv.shape, 0)
                    + ci * _CHUNK)
            rmv_parts.append(rmv)
            rmi_parts.append(rmi_rel + offs)
        rmv0.append(jnp.concatenate(rmv_parts, axis=0))
        rmi0.append(jnp.concatenate(rmi_parts, axis=0))

    wiota = lax.broadcasted_iota(jnp.int32, (1, _HM), 1)
    w80v = wiota * _N_CLASSES
    ciota = lax.broadcasted_iota(jnp.int32, (_N_CLASSES, _HM), 0)
    hiota = lax.broadcasted_iota(jnp.int32, (_N_CLASSES, _HM), 1)
    liota2 = lax.broadcasted_iota(jnp.int32, (1, 2), 1)

    def step(i, carry):
        new_carry = []
        for b in range(_G):
            rmv, rmi = carry[b]
            m = jnp.max(rmv)
            f = jnp.min(jnp.where(rmv == m, rmi, _BIG))
            c = f % _N_CLASSES
            sp = f // _N_CLASSES
            w = sp % _HM
            h = sp // _HM
            r = c * _HM + h
            row = s_refs[b][pl.ds(r, 1), :]
            newrow = jnp.where(wiota == w, _NEG, row)
            s_refs[b][pl.ds(r, 1), :] = newrow
            nm = jnp.max(newrow)
            ri = jnp.min(jnp.where(newrow == nm, w80v, _BIG)) \
                + h * (_HM * _N_CLASSES) + c
            hit = (ciota == c) & (hiota == h)
            sel_refs[b][pl.ds(i, 1), :] = jnp.where(
                liota2 == 0, m, f.astype(jnp.float32))
            new_carry.append((jnp.where(hit, nm, rmv), jnp.where(hit, ri, rmi)))
        return tuple(new_carry)

    lax.fori_loop(0, _K, step, tuple(zip(rmv0, rmi0)), unroll=2)

    kiota = lax.broadcasted_iota(jnp.int32, (_K, _HM), 1)
    liota8 = lax.broadcasted_iota(jnp.int32, (_K, 8), 1)
    for b in range(_G):
        sel = sel_refs[b][:, :]
        mv = sel[:, 0:1]
        f = sel[:, 1:2].astype(jnp.int32)
        c = f % _N_CLASSES
        sp = f // _N_CLASSES
        w = sp % _HM
        h = sp // _HM
        oh_h = (h == kiota).astype(jnp.float32)
        oh_w = (w == kiota).astype(jnp.float32)
        g = []
        for j in range(4):
            cj = y_ref[b, _N_CLASSES + j, :, :]
            rows = jnp.dot(oh_h, cj, preferred_element_type=jnp.float32)
            g.append(jnp.sum(rows * oh_w, axis=1, keepdims=True))
        xs = w.astype(jnp.float32)
        ys = h.astype(jnp.float32)
        cls = c.astype(jnp.float32) + 1.0
        x1 = (_DOWN * xs - g[0]) / _SIZE
        y1 = (_DOWN * ys - g[1]) / _SIZE
        x2 = (_DOWN * xs + g[2]) / _SIZE
        y2 = (_DOWN * ys + g[3]) / _SIZE
        det = jnp.where(
            liota8 == 0, cls,
            jnp.where(liota8 == 1, mv,
                      jnp.where(liota8 == 2, x1,
                                jnp.where(liota8 == 3, y1,
                                          jnp.where(liota8 == 4, x2,
                                                    jnp.where(liota8 == 5, y2,
                                                              jnp.where(liota8 == 6, ys, xs)))))))
        out_ref[b, :, :] = det


@jax.jit
def kernel(y_pred):
    batch = y_pred.shape[0]
    assert batch % _G == 0
    return pl.pallas_call(
        _nms_topk_body,
        grid=(batch // _G,),
        in_specs=[pl.BlockSpec((_G, _N_CLASSES + 4, _HM, _HM),
                               lambda b: (b, 0, 0, 0))],
        out_specs=pl.BlockSpec((_G, _K, 8), lambda b: (b, 0, 0)),
        out_shape=jax.ShapeDtypeStruct((batch, _K, 8), jnp.float32),
        compiler_params=pltpu.CompilerParams(
            dimension_semantics=("parallel",)),
        scratch_shapes=(
            [pltpu.VMEM((_ROWS, _HM), jnp.float32) for _ in range(_G)]
            + [pltpu.VMEM((_K, 2), jnp.float32) for _ in range(_G)]),
    )(y_pred)


# speedup vs baseline: 1.0246x; 1.0246x over previous
import jax
import jax.numpy as jnp
from jax import lax
from jax.experimental import pallas as pl
from jax.experimental.pallas import tpu as pltpu

_N_CLASSES = 80
_HM = 128
_K = 100
_DOWN = 4.0
_SIZE = 512.0
_CHUNK = 8
_G = 2
_ROWS = _N_CLASSES * _HM
_NEG = float("-inf")
_BIG = 2 ** 30


def _nms_topk_body(y_ref, out_ref, *scratch):
    s_refs = scratch[:_G]
    sel_refs = scratch[_G:]
    rmv0 = []
    rmi0 = []
    for b in range(_G):
        rmv_parts = []
        rmi_parts = []
        for ci in range(_N_CLASSES // _CHUNK):
            x = y_ref[b, ci * _CHUNK:(ci + 1) * _CHUNK, :, :]
            pad_h = jnp.full((_CHUNK, 1, _HM), _NEG, jnp.float32)
            up = jnp.concatenate([x[:, 1:, :], pad_h], axis=1)
            dn = jnp.concatenate([pad_h, x[:, :-1, :]], axis=1)
            v = jnp.maximum(jnp.maximum(x, up), dn)
            pad_w = jnp.full((_CHUNK, _HM, 1), _NEG, jnp.float32)
            lf = jnp.concatenate([v[:, :, 1:], pad_w], axis=2)
            rt = jnp.concatenate([pad_w, v[:, :, :-1]], axis=2)
            hmax = jnp.maximum(jnp.maximum(v, lf), rt)
            sc = jnp.where(hmax == x, x, 0.0)
            s_refs[b][pl.ds(ci * _CHUNK * _HM, _CHUNK * _HM), :] = \
                sc.reshape(_CHUNK * _HM, _HM)
            rmv = jnp.max(sc, axis=2)
            w80 = lax.broadcasted_iota(jnp.int32, sc.shape, 2) * _N_CLASSES
            rmi_rel = jnp.min(
                jnp.where(sc == rmv[:, :, None], w80, _BIG), axis=2)
            offs = (lax.broadcasted_iota(jnp.int32, rmv.shape, 1)
                    * (_HM * _N_CLASSES)
                    + lax.broadcasted_iota(jnp.int32, rmv.shape, 0)
                    + ci * _CHUNK)
            rmv_parts.append(rmv)
            rmi_parts.append(rmi_rel + offs)
        rmv0.append(jnp.concatenate(rmv_parts, axis=0))
        rmi0.append(jnp.concatenate(rmi_parts, axis=0))

    wiota = lax.broadcasted_iota(jnp.int32, (1, _HM), 1)
    w80v = wiota * _N_CLASSES
    ciota = lax.broadcasted_iota(jnp.int32, (_N_CLASSES, _HM), 0)
    hiota = lax.broadcasted_iota(jnp.int32, (_N_CLASSES, _HM), 1)
    liota2 = lax.broadcasted_iota(jnp.int32, (1, 2), 1)

    def step(i, carry):
        new_carry = []
        for b in range(_G):
            rmv, rmi = carry[b]
            m = jnp.max(rmv)
            f = jnp.min(jnp.where(rmv == m, rmi, _BIG))
            c = f % _N_CLASSES
            sp = f // _N_CLASSES
            w = sp % _HM
            h = sp // _HM
            r = c * _HM + h
            row = s_refs[b][pl.ds(r, 1), :]
            newrow = jnp.where(wiota == w, _NEG, row)
            s_refs[b][pl.ds(r, 1), :] = newrow
            nm = jnp.max(newrow)
            ri = jnp.min(jnp.where(newrow == nm, w80v, _BIG)) \
                + h * (_HM * _N_CLASSES) + c
            hit = (ciota == c) & (hiota == h)
            sel_refs[b][pl.ds(i, 1), :] = jnp.where(
                liota2 == 0, m, f.astype(jnp.float32))
            new_carry.append((jnp.where(hit, nm, rmv), jnp.where(hit, ri, rmi)))
        return tuple(new_carry)

    lax.fori_loop(0, _K, step, tuple(zip(rmv0, rmi0)), unroll=4)

    kiota = lax.broadcasted_iota(jnp.int32, (_K, _HM), 1)
    liota8 = lax.broadcasted_iota(jnp.int32, (_K, 8), 1)
    for b in range(_G):
        sel = sel_refs[b][:, :]
        mv = sel[:, 0:1]
        f = sel[:, 1:2].astype(jnp.int32)
        c = f % _N_CLASSES
        sp = f // _N_CLASSES
        w = sp % _HM
        h = sp // _HM
        oh_h = (h == kiota).astype(jnp.float32)
        oh_w = (w == kiota).astype(jnp.float32)
        g = []
        for j in range(4):
            cj = y_ref[b, _N_CLASSES + j, :, :]
            rows = jnp.dot(oh_h, cj, preferred_element_type=jnp.float32)
            g.append(jnp.sum(rows * oh_w, axis=1, keepdims=True))
        xs = w.astype(jnp.float32)
        ys = h.astype(jnp.float32)
        cls = c.astype(jnp.float32) + 1.0
        x1 = (_DOWN * xs - g[0]) / _SIZE
        y1 = (_DOWN * ys - g[1]) / _SIZE
        x2 = (_DOWN * xs + g[2]) / _SIZE
        y2 = (_DOWN * ys + g[3]) / _SIZE
        det = jnp.where(
            liota8 == 0, cls,
            jnp.where(liota8 == 1, mv,
                      jnp.where(liota8 == 2, x1,
                                jnp.where(liota8 == 3, y1,
                                          jnp.where(liota8 == 4, x2,
                                                    jnp.where(liota8 == 5, y2,
                                                              jnp.where(liota8 == 6, ys, xs)))))))
        out_ref[b, :, :] = det


@jax.jit
def kernel(y_pred):
    batch = y_pred.shape[0]
    assert batch % _G == 0
    return pl.pallas_call(
        _nms_topk_body,
        grid=(batch // _G,),
        in_specs=[pl.BlockSpec((_G, _N_CLASSES + 4, _HM, _HM),
                               lambda b: (b, 0, 0, 0))],
        out_specs=pl.BlockSpec((_G, _K, 8), lambda b: (b, 0, 0)),
        out_shape=jax.ShapeDtypeStruct((batch, _K, 8), jnp.float32),
        compiler_params=pltpu.CompilerParams(
            dimension_semantics=("parallel",)),
        scratch_shapes=(
            [pltpu.VMEM((_ROWS, _HM), jnp.float32) for _ in range(_G)]
            + [pltpu.VMEM((_K, 2), jnp.float32) for _ in range(_G)]),
    )(y_pred)
